# Initial kernel scaffold; baseline (speedup 1.0000x reference)
import jax
import jax.numpy as jnp
from jax.experimental import pallas as pl


def kernel(x, edge_index, W1, b1, W2, b2, W3, b3):
    raise NotImplementedError("write your pallas kernel here")


# speedup vs baseline: 7.5566x; 7.5566x over previous
import functools

import jax
import jax.numpy as jnp
from jax import lax
from jax.experimental import pallas as pl
from jax.experimental.pallas import tpu as pltpu
from jax.experimental.pallas import tpu_sc as plsc

N = 10000
E = 320000
D = 128
H = 128
C = 40
CP = 128
NPAD = 10240
NC = 2
NS = 16
NW = NC * NS
LANES = 16

KE = 80
ECH = E // KE
ROWS_PER_TILE = NPAD // NS

_MESH = plsc.VectorSubcoreMesh(
    core_axis_name="c", subcore_axis_name="s", num_cores=NC, num_subcores=NS
)


def _zero_fill_2d(ref, rows, cols):
    zv = jnp.zeros((LANES,), jnp.float32)

    def body(i, _):
        r = i // (cols // LANES)
        col = (i % (cols // LANES)) * LANES
        ref[r, pl.ds(col, LANES)] = zv
        return 0

    lax.fori_loop(0, rows * (cols // LANES), body, 0)


@functools.partial(
    pl.kernel,
    out_type=jax.ShapeDtypeStruct((NC, 1, NPAD), jnp.float32),
    mesh=_MESH,
    scratch_types=[
        pltpu.VMEM_SHARED((NPAD,), jnp.float32),
        pltpu.VMEM((ECH // NS, KE), jnp.int32),
        pltpu.VMEM((KE,), jnp.float32),
        pltpu.VMEM((ROWS_PER_TILE,), jnp.float32),
    ],
)
def _degrees(ei_hbm, deg_hbm, acc, idx_v, ones_v, zb_v):
    c = lax.axis_index("c")
    s = lax.axis_index("s")

    def fill(i, _):
        ones_v[pl.ds(i * LANES, LANES)] = jnp.ones((LANES,), jnp.float32)
        return 0

    lax.fori_loop(0, KE // LANES, fill, 0)

    def fillz(i, _):
        zb_v[pl.ds(i * LANES, LANES)] = jnp.zeros((LANES,), jnp.float32)
        return 0

    lax.fori_loop(0, ROWS_PER_TILE // LANES, fillz, 0)

    pltpu.sync_copy(zb_v, acc.at[pl.ds(s * ROWS_PER_TILE, ROWS_PER_TILE)])
    plsc.subcore_barrier()

    nrows = ECH // NS
    pltpu.sync_copy(ei_hbm.at[c, s], idx_v)

    def chunk(j, _):
        pltpu.sync_copy(ones_v, acc.at[idx_v.at[j]], add=True)
        return 0

    lax.fori_loop(0, nrows, chunk, 0)
    plsc.subcore_barrier()

    pltpu.sync_copy(
        acc.at[pl.ds(s * ROWS_PER_TILE, ROWS_PER_TILE)],
        deg_hbm.at[c, 0, pl.ds(s * ROWS_PER_TILE, ROWS_PER_TILE)],
    )


def _make_agg(width):
    ept = E // NW
    nrows = ept // KE

    @functools.partial(
        pl.kernel,
        out_type=jax.ShapeDtypeStruct((NC, NPAD, width), jnp.float32),
        mesh=_MESH,
        scratch_types=[
            pltpu.VMEM_SHARED((NPAD, width), jnp.float32),
            pltpu.VMEM((nrows, KE), jnp.int32),
            pltpu.VMEM((nrows, KE), jnp.int32),
            pltpu.VMEM((KE, width), jnp.float32),
            pltpu.SemaphoreType.DMA,
        ],
    )
    def agg(h_hbm, src_hbm, dst_hbm, out_hbm, acc, si_v, di_v, rows_v, sem):
        c = lax.axis_index("c")
        s = lax.axis_index("s")
        wid = c * NS + s

        _zero_fill_2d(rows_v, KE, width)

        def zc(k, _):
            pltpu.sync_copy(
                rows_v, acc.at[pl.ds(s * ROWS_PER_TILE + k * KE, KE), :]
            )
            return 0

        lax.fori_loop(0, ROWS_PER_TILE // KE, zc, 0)
        plsc.subcore_barrier()

        pltpu.sync_copy(src_hbm.at[wid], si_v)
        pltpu.sync_copy(dst_hbm.at[wid], di_v)

        def chunk(j, _):
            pltpu.async_copy(h_hbm.at[si_v.at[j]], rows_v, sem).wait()
            pltpu.sync_copy(rows_v, acc.at[di_v.at[j]], add=True)
            return 0

        lax.fori_loop(0, nrows, chunk, 0)
        plsc.subcore_barrier()

        pltpu.sync_copy(
            acc.at[pl.ds(s * ROWS_PER_TILE, ROWS_PER_TILE), :],
            out_hbm.at[c, pl.ds(s * ROWS_PER_TILE, ROWS_PER_TILE), :],
        )

    return agg


_agg_h = _make_agg(H)
_agg_c = _agg_h


_BLK = 1024
_GRID = NPAD // _BLK


def _mm_scale_body(x_ref, w_ref, dout_ref, o_ref):
    rs = lax.rsqrt(jnp.maximum(dout_ref[...], 1.0))
    o_ref[...] = jnp.dot(
        x_ref[...], w_ref[...], preferred_element_type=jnp.float32
    ) * rs


def _mm_scale(xp, w, deg_out):
    return pl.pallas_call(
        _mm_scale_body,
        grid=(_GRID,),
        in_specs=[
            pl.BlockSpec((_BLK, D), lambda i: (i, 0)),
            pl.BlockSpec((D, H), lambda i: (0, 0)),
            pl.BlockSpec((_BLK, 1), lambda i: (i, 0)),
        ],
        out_specs=pl.BlockSpec((_BLK, H), lambda i: (i, 0)),
        out_shape=jax.ShapeDtypeStruct((NPAD, H), jnp.float32),
    )(xp, w, deg_out)


def _combine_mm_body(p_ref, b_ref, din_ref, dout_ref, w_ref, o_ref):
    rs_in = lax.rsqrt(jnp.maximum(din_ref[...], 1.0))
    rs_out = lax.rsqrt(jnp.maximum(dout_ref[...], 1.0))
    h = (p_ref[0] + p_ref[1]) * rs_in + b_ref[...]
    h = jnp.maximum(h, 0.0)
    o_ref[...] = jnp.dot(
        h, w_ref[...], preferred_element_type=jnp.float32
    ) * rs_out


def _combine_mm(p, b, deg_in, deg_out, w):
    wout = w.shape[1]
    return pl.pallas_call(
        _combine_mm_body,
        grid=(_GRID,),
        in_specs=[
            pl.BlockSpec((NC, _BLK, H), lambda i: (0, i, 0)),
            pl.BlockSpec((1, H), lambda i: (0, 0)),
            pl.BlockSpec((_BLK, 1), lambda i: (i, 0)),
            pl.BlockSpec((_BLK, 1), lambda i: (i, 0)),
            pl.BlockSpec((H, wout), lambda i: (0, 0)),
        ],
        out_specs=pl.BlockSpec((_BLK, wout), lambda i: (i, 0)),
        out_shape=jax.ShapeDtypeStruct((NPAD, wout), jnp.float32),
    )(p, b, deg_in, deg_out, w)


def _final_body(p_ref, b_ref, din_ref, o_ref):
    rs_in = lax.rsqrt(jnp.maximum(din_ref[...], 1.0))
    o_ref[...] = (p_ref[0] + p_ref[1]) * rs_in + b_ref[...]


def _final(p, b, deg_in):
    return pl.pallas_call(
        _final_body,
        grid=(_GRID,),
        in_specs=[
            pl.BlockSpec((NC, _BLK, CP), lambda i: (0, i, 0)),
            pl.BlockSpec((1, CP), lambda i: (0, 0)),
            pl.BlockSpec((_BLK, 1), lambda i: (i, 0)),
        ],
        out_specs=pl.BlockSpec((_BLK, CP), lambda i: (i, 0)),
        out_shape=jax.ShapeDtypeStruct((NPAD, CP), jnp.float32),
    )(p, b, deg_in)


def kernel(x, edge_index, W1, b1, W2, b2, W3, b3):
    xp = jnp.zeros((NPAD, D), jnp.float32).at[:N].set(x)
    ei4 = edge_index.reshape(2, NS, ECH // NS, KE)
    src3 = edge_index[0].reshape(NW, E // NW // KE, KE)
    dst3 = edge_index[1].reshape(NW, E // NW // KE, KE)
    W3p = jnp.pad(W3, ((0, 0), (0, CP - C)))
    b3p = jnp.pad(b3, (0, CP - C))

    deg = _degrees(ei4)
    deg_out = deg[0].reshape(NPAD, 1)
    deg_in = deg[1].reshape(NPAD, 1)

    h1 = _mm_scale(xp, W1, deg_out)
    p1 = _agg_h(h1, src3, dst3)
    h2 = _combine_mm(p1, b1.reshape(1, H), deg_in, deg_out, W2)
    p2 = _agg_h(h2, src3, dst3)
    h3 = _combine_mm(p2, b2.reshape(1, H), deg_in, deg_out, W3p)
    p3 = _agg_c(h3, src3, dst3)
    out = _final(p3, b3p.reshape(1, CP), deg_in)
    return out[:N, :C]


# speedup vs baseline: 10.8129x; 1.4309x over previous
import functools

import jax
import jax.numpy as jnp
from jax import lax
from jax.experimental import pallas as pl
from jax.experimental.pallas import tpu as pltpu
from jax.experimental.pallas import tpu_sc as plsc

N = 10000
E = 320000
D = 128
H = 128
C = 40
CP = 128
NPAD = 10240
NC = 2
NS = 16
NW = NC * NS
LANES = 16

KD = 80
DCH = E // KD
KE = 80
ROWS_PER_TILE = NPAD // NS

_MESH = plsc.VectorSubcoreMesh(
    core_axis_name="c", subcore_axis_name="s", num_cores=NC, num_subcores=NS
)


def _zero_fill_2d(ref, rows, cols):
    zv = jnp.zeros((LANES,), jnp.float32)

    def body(i, _):
        r = i // (cols // LANES)
        col = (i % (cols // LANES)) * LANES
        ref[r, pl.ds(col, LANES)] = zv
        return 0

    lax.fori_loop(0, rows * (cols // LANES), body, 0)


@functools.partial(
    pl.kernel,
    out_type=jax.ShapeDtypeStruct((NC, 1, NPAD), jnp.float32),
    mesh=_MESH,
    scratch_types=[
        pltpu.VMEM_SHARED((NPAD,), jnp.float32),
        pltpu.VMEM((DCH // NS, KD), jnp.int32),
        pltpu.VMEM((KD,), jnp.float32),
        pltpu.VMEM((ROWS_PER_TILE,), jnp.float32),
    ],
)
def _degrees(ei_hbm, deg_hbm, acc, idx_v, ones_v, zb_v):
    c = lax.axis_index("c")
    s = lax.axis_index("s")

    def fill(i, _):
        ones_v[pl.ds(i * LANES, LANES)] = jnp.ones((LANES,), jnp.float32)
        return 0

    lax.fori_loop(0, KD // LANES, fill, 0)

    def fillz(i, _):
        zb_v[pl.ds(i * LANES, LANES)] = jnp.zeros((LANES,), jnp.float32)
        return 0

    lax.fori_loop(0, ROWS_PER_TILE // LANES, fillz, 0)

    pltpu.sync_copy(zb_v, acc.at[pl.ds(s * ROWS_PER_TILE, ROWS_PER_TILE)])
    plsc.subcore_barrier()

    nrows = DCH // NS
    pltpu.sync_copy(ei_hbm.at[c, s], idx_v)

    def chunk(j, _):
        pltpu.sync_copy(ones_v, acc.at[idx_v.at[j]], add=True)
        return 0

    lax.fori_loop(0, nrows, chunk, 0)
    plsc.subcore_barrier()

    pltpu.sync_copy(
        acc.at[pl.ds(s * ROWS_PER_TILE, ROWS_PER_TILE)],
        deg_hbm.at[c, 0, pl.ds(s * ROWS_PER_TILE, ROWS_PER_TILE)],
    )


NCHK = (E // NW) // KE


def _make_agg(width):
    @functools.partial(
        pl.kernel,
        out_type=jax.ShapeDtypeStruct((NC, NPAD, width), jnp.float32),
        mesh=_MESH,
        scratch_types=[
            pltpu.VMEM_SHARED((NPAD, width), jnp.float32),
            pltpu.VMEM((NCHK, KE), jnp.int32),
            pltpu.VMEM((KE, width), jnp.float32),
            pltpu.VMEM((KE, width), jnp.float32),
            pltpu.VMEM((1, KE), jnp.int32),
            pltpu.VMEM((1, KE), jnp.int32),
            pltpu.SemaphoreType.DMA,
            pltpu.SemaphoreType.DMA,
            pltpu.SemaphoreType.DMA,
        ],
    )
    def agg(h_hbm, src_hbm, dst_hbm, out_hbm, acc, si_v, r0, r1, d0, d1,
            sg0, sg1, sd):
        c = lax.axis_index("c")
        s = lax.axis_index("s")
        wid = c * NS + s

        def gat(j, rv, sem):
            return pltpu.make_async_copy(h_hbm.at[si_v.at[j]], rv, sem)

        def sca(rv, dv):
            pltpu.sync_copy(rv, acc.at[dv.at[0]], add=True)

        def dpre(j, dv):
            pltpu.make_async_copy(
                dst_hbm.at[wid * NCHK + j], dv, sd
            ).start()

        def dwait(dv):
            pltpu.make_async_copy(dst_hbm.at[0], dv, sd).wait()

        _zero_fill_2d(r0, KE, width)

        def zc(k, _):
            pltpu.sync_copy(
                r0, acc.at[pl.ds(s * ROWS_PER_TILE + k * KE, KE), :]
            )
            return 0

        lax.fori_loop(0, ROWS_PER_TILE // KE, zc, 0)
        plsc.subcore_barrier()

        pltpu.sync_copy(src_hbm.at[wid], si_v)
        pltpu.sync_copy(dst_hbm.at[wid * NCHK], d0)

        gat(0, r0, sg0).start()
        dpre(1, d1)

        def body(i2, _):
            j1 = 2 * i2 + 1
            gat(j1, r1, sg1).start()
            gat(j1 - 1, r0, sg0).wait()
            dwait(d1)
            sca(r0, d0)
            dpre(j1 + 1, d0)
            j2 = j1 + 1
            gat(j2, r0, sg0).start()
            gat(j1, r1, sg1).wait()
            dwait(d0)
            sca(r1, d1)
            dpre(jnp.minimum(j2 + 1, NCHK - 1), d1)
            return 0

        lax.fori_loop(0, (NCHK - 1) // 2, body, 0)
        gat(NCHK - 1, r0, sg0).wait()
        dwait(d1)
        sca(r0, d0)
        plsc.subcore_barrier()

        pltpu.sync_copy(
            acc.at[pl.ds(s * ROWS_PER_TILE, ROWS_PER_TILE), :],
            out_hbm.at[c, pl.ds(s * ROWS_PER_TILE, ROWS_PER_TILE), :],
        )

    return agg


_agg_h = _make_agg(H)
_agg_c = _agg_h


_BLK = 1024
_GRID = NPAD // _BLK


def _mm_scale_body(x_ref, w_ref, dout_ref, o_ref):
    rs = lax.rsqrt(jnp.maximum(dout_ref[...], 1.0))
    o_ref[...] = jnp.dot(
        x_ref[...], w_ref[...], preferred_element_type=jnp.float32
    ) * rs


def _mm_scale(xp, w, deg_out):
    return pl.pallas_call(
        _mm_scale_body,
        grid=(_GRID,),
        in_specs=[
            pl.BlockSpec((_BLK, D), lambda i: (i, 0)),
            pl.BlockSpec((D, H), lambda i: (0, 0)),
            pl.BlockSpec((_BLK, 1), lambda i: (i, 0)),
        ],
        out_specs=pl.BlockSpec((_BLK, H), lambda i: (i, 0)),
        out_shape=jax.ShapeDtypeStruct((NPAD, H), jnp.float32),
    )(xp, w, deg_out)


def _combine_mm_body(p_ref, b_ref, din_ref, dout_ref, w_ref, o_ref):
    rs_in = lax.rsqrt(jnp.maximum(din_ref[...], 1.0))
    rs_out = lax.rsqrt(jnp.maximum(dout_ref[...], 1.0))
    h = (p_ref[0] + p_ref[1]) * rs_in + b_ref[...]
    h = jnp.maximum(h, 0.0)
    o_ref[...] = jnp.dot(
        h, w_ref[...], preferred_element_type=jnp.float32
    ) * rs_out


def _combine_mm(p, b, deg_in, deg_out, w):
    wout = w.shape[1]
    return pl.pallas_call(
        _combine_mm_body,
        grid=(_GRID,),
        in_specs=[
            pl.BlockSpec((NC, _BLK, H), lambda i: (0, i, 0)),
            pl.BlockSpec((1, H), lambda i: (0, 0)),
            pl.BlockSpec((_BLK, 1), lambda i: (i, 0)),
            pl.BlockSpec((_BLK, 1), lambda i: (i, 0)),
            pl.BlockSpec((H, wout), lambda i: (0, 0)),
        ],
        out_specs=pl.BlockSpec((_BLK, wout), lambda i: (i, 0)),
        out_shape=jax.ShapeDtypeStruct((NPAD, wout), jnp.float32),
    )(p, b, deg_in, deg_out, w)


def _final_body(p_ref, b_ref, din_ref, o_ref):
    rs_in = lax.rsqrt(jnp.maximum(din_ref[...], 1.0))
    o_ref[...] = (p_ref[0] + p_ref[1]) * rs_in + b_ref[...]


def _final(p, b, deg_in):
    return pl.pallas_call(
        _final_body,
        grid=(_GRID,),
        in_specs=[
            pl.BlockSpec((NC, _BLK, CP), lambda i: (0, i, 0)),
            pl.BlockSpec((1, CP), lambda i: (0, 0)),
            pl.BlockSpec((_BLK, 1), lambda i: (i, 0)),
        ],
        out_specs=pl.BlockSpec((_BLK, CP), lambda i: (i, 0)),
        out_shape=jax.ShapeDtypeStruct((NPAD, CP), jnp.float32),
    )(p, b, deg_in)


def kernel(x, edge_index, W1, b1, W2, b2, W3, b3):
    xp = jnp.zeros((NPAD, D), jnp.float32).at[:N].set(x)
    ei4 = edge_index.reshape(2, NS, DCH // NS, KD)
    src3 = edge_index[0].reshape(NW, NCHK, KE)
    dst3 = edge_index[1].reshape(NW * NCHK, 1, KE)
    W3p = jnp.pad(W3, ((0, 0), (0, CP - C)))
    b3p = jnp.pad(b3, (0, CP - C))

    deg = _degrees(ei4)
    deg_out = deg[0].reshape(NPAD, 1)
    deg_in = deg[1].reshape(NPAD, 1)

    h1 = _mm_scale(xp, W1, deg_out)
    p1 = _agg_h(h1, src3, dst3)
    h2 = _combine_mm(p1, b1.reshape(1, H), deg_in, deg_out, W2)
    p2 = _agg_h(h2, src3, dst3)
    h3 = _combine_mm(p2, b2.reshape(1, H), deg_in, deg_out, W3p)
    p3 = _agg_c(h3, src3, dst3)
    out = _final(p3, b3p.reshape(1, CP), deg_in)
    return out[:N, :C]
